# baseline (speedup 1.0000x reference)
import functools

import jax
import jax.numpy as jnp
from jax import lax
from jax.experimental import pallas as pl
from jax.experimental.pallas import tpu as pltpu
from jax.experimental.pallas import tpu_sc as plsc

NC = 2
NS = 16
NW = NC * NS


def _sc_gather_body(bpw, uidx_hbm, vidx_hbm, w_hbm, h_hbm, u_out, v_out,
                    uidx_v, vidx_v, urows_v, vrows_v, sem):
    wid = lax.axis_index("s") * NC + lax.axis_index("c")
    base = wid * bpw
    pltpu.sync_copy(uidx_hbm.at[pl.ds(base, bpw)], uidx_v)
    pltpu.sync_copy(vidx_hbm.at[pl.ds(base, bpw)], vidx_v)
    cp_u = pltpu.async_copy(w_hbm.at[uidx_v], urows_v, sem)
    cp_v = pltpu.async_copy(h_hbm.at[vidx_v], vrows_v, sem)
    cp_u.wait()
    cp_v.wait()
    pltpu.sync_copy(urows_v, u_out.at[pl.ds(base, bpw)])
    pltpu.sync_copy(vrows_v, v_out.at[pl.ds(base, bpw)])


@functools.lru_cache(maxsize=None)
def _make_sc_gather(batch, emb_k):
    bpw = batch // NW
    mesh = plsc.VectorSubcoreMesh(core_axis_name="c", subcore_axis_name="s")
    return pl.kernel(
        functools.partial(_sc_gather_body, bpw),
        mesh=mesh,
        out_type=[
            jax.ShapeDtypeStruct((batch, emb_k), jnp.float32),
            jax.ShapeDtypeStruct((batch, emb_k), jnp.float32),
        ],
        scratch_types=[
            pltpu.VMEM((bpw,), jnp.int32),
            pltpu.VMEM((bpw,), jnp.int32),
            pltpu.VMEM((bpw, emb_k), jnp.float32),
            pltpu.VMEM((bpw, emb_k), jnp.float32),
            pltpu.SemaphoreType.DMA,
        ],
        compiler_params=pltpu.CompilerParams(use_tc_tiling_on_sc=False),
    )


def _mlp_body(u_ref, v_ref, a1_ref, a2_ref, b_ref, w2_ref, o_ref):
    h = jnp.dot(u_ref[...], a1_ref[...], preferred_element_type=jnp.float32)
    h = h + jnp.dot(v_ref[...], a2_ref[...], preferred_element_type=jnp.float32)
    h = jnp.maximum(h + b_ref[...], 0.0)
    o_ref[...] = jnp.dot(h, w2_ref[...], preferred_element_type=jnp.float32)


@functools.lru_cache(maxsize=None)
def _make_mlp(batch, emb_k, blk):
    grid = (batch // blk,)
    return pl.pallas_call(
        _mlp_body,
        grid=grid,
        in_specs=[
            pl.BlockSpec((blk, emb_k), lambda i: (i, 0)),
            pl.BlockSpec((blk, emb_k), lambda i: (i, 0)),
            pl.BlockSpec((emb_k, emb_k), lambda i: (0, 0)),
            pl.BlockSpec((emb_k, emb_k), lambda i: (0, 0)),
            pl.BlockSpec((1, emb_k), lambda i: (0, 0)),
            pl.BlockSpec((emb_k, 1), lambda i: (0, 0)),
        ],
        out_specs=pl.BlockSpec((blk, 1), lambda i: (i, 0)),
        out_shape=jax.ShapeDtypeStruct((batch, 1), jnp.float32),
    )


def kernel(x, W, H, lin1_w, lin1_b, lin2_w):
    batch = x.shape[0]
    emb_k = W.shape[1]
    u_idx = x[:, 0].astype(jnp.int32)
    v_idx = x[:, 1].astype(jnp.int32)
    U, V = _make_sc_gather(batch, emb_k)(u_idx, v_idx, W, H)
    a1 = lin1_w[:, :emb_k].T
    a2 = lin1_w[:, emb_k:].T
    b = lin1_b.reshape(1, emb_k)
    w2 = lin2_w.T
    return _make_mlp(batch, emb_k, 2048)(U, V, a1, a2, b, w2)


# speedup vs baseline: 1.0098x; 1.0098x over previous
import functools

import jax
import jax.numpy as jnp
from jax import lax
from jax.experimental import pallas as pl
from jax.experimental.pallas import tpu as pltpu
from jax.experimental.pallas import tpu_sc as plsc

NC = 2
NS = 16
NW = NC * NS
CHUNK = 128


def _sc_gather_body(bpw, uhi_hbm, ulo_hbm, vhi_hbm, vlo_hbm, w_hbm, h_hbm,
                    u_out, v_out, uhi_v, ulo_v, vhi_v, vlo_v, super_v, rows_v,
                    sem):
    wid = lax.axis_index("s") * NC + lax.axis_index("c")
    base = wid * bpw
    pltpu.sync_copy(uhi_hbm.at[pl.ds(base, bpw)], uhi_v)
    pltpu.sync_copy(ulo_hbm.at[pl.ds(base, bpw)], ulo_v)
    pltpu.sync_copy(vhi_hbm.at[pl.ds(base, bpw)], vhi_v)
    pltpu.sync_copy(vlo_hbm.at[pl.ds(base, bpw)], vlo_v)

    n_chunks = bpw // CHUNK
    rows_per_chunk = CHUNK // 8

    def do_table(hi_v, lo_v, tab_hbm, out_hbm):
        def chunk_body(c, _):
            off = c * CHUNK
            pltpu.async_copy(
                tab_hbm.at[hi_v.at[pl.ds(off, CHUNK)]], super_v, sem
            ).wait()
            for g in range(CHUNK // 16):
                lo_vec = lo_v[pl.ds(off + g * 16, 16)]
                for s in range(16):
                    i = g * 16 + s
                    sub = lo_vec[s] * 16
                    rows_v[i // 8, pl.ds((i % 8) * 16, 16)] = (
                        super_v[i, pl.ds(sub, 16)]
                    )
            row0 = pl.multiple_of((base + off) // 8, rows_per_chunk)
            pltpu.sync_copy(rows_v, out_hbm.at[pl.ds(row0, rows_per_chunk)])
            return ()

        lax.fori_loop(0, n_chunks, chunk_body, ())

    do_table(uhi_v, ulo_v, w_hbm, u_out)
    do_table(vhi_v, vlo_v, h_hbm, v_out)


@functools.lru_cache(maxsize=None)
def _make_sc_gather(batch, emb_k):
    bpw = batch // NW
    mesh = plsc.VectorSubcoreMesh(core_axis_name="c", subcore_axis_name="s")
    return pl.kernel(
        functools.partial(_sc_gather_body, bpw),
        mesh=mesh,
        out_type=[
            jax.ShapeDtypeStruct((batch // 8, 8 * emb_k), jnp.float32),
            jax.ShapeDtypeStruct((batch // 8, 8 * emb_k), jnp.float32),
        ],
        scratch_types=[
            pltpu.VMEM((bpw,), jnp.int32),
            pltpu.VMEM((bpw,), jnp.int32),
            pltpu.VMEM((bpw,), jnp.int32),
            pltpu.VMEM((bpw,), jnp.int32),
            pltpu.VMEM((CHUNK, 8 * emb_k), jnp.float32),
            pltpu.VMEM((CHUNK // 8, 8 * emb_k), jnp.float32),
            pltpu.SemaphoreType.DMA,
        ],
    )


def _mlp_body(u_ref, v_ref, a1_ref, a2_ref, b_ref, w2_ref, o_ref):
    h = jnp.dot(u_ref[...], a1_ref[...], preferred_element_type=jnp.float32)
    h = h + jnp.dot(v_ref[...], a2_ref[...], preferred_element_type=jnp.float32)
    h = jnp.maximum(h + b_ref[...], 0.0)
    o_ref[...] = jnp.dot(h, w2_ref[...], preferred_element_type=jnp.float32)


@functools.lru_cache(maxsize=None)
def _make_mlp(rows, width, blk):
    grid = (rows // blk,)
    return pl.pallas_call(
        _mlp_body,
        grid=grid,
        in_specs=[
            pl.BlockSpec((blk, width), lambda i: (i, 0)),
            pl.BlockSpec((blk, width), lambda i: (i, 0)),
            pl.BlockSpec((width, width), lambda i: (0, 0)),
            pl.BlockSpec((width, width), lambda i: (0, 0)),
            pl.BlockSpec((1, width), lambda i: (0, 0)),
            pl.BlockSpec((width, 8), lambda i: (0, 0)),
        ],
        out_specs=pl.BlockSpec((blk, 8), lambda i: (i, 0)),
        out_shape=jax.ShapeDtypeStruct((rows, 8), jnp.float32),
    )


def _block_diag8(m):
    k, n = m.shape
    eye = jnp.eye(8, dtype=m.dtype)
    return (eye[:, None, :, None] * m[None, :, None, :]).reshape(8 * k, 8 * n)


def kernel(x, W, H, lin1_w, lin1_b, lin2_w):
    batch = x.shape[0]
    emb_k = W.shape[1]
    u_idx = x[:, 0].astype(jnp.int32)
    v_idx = x[:, 1].astype(jnp.int32)
    wp = W.reshape(W.shape[0] // 8, 8 * emb_k)
    hp = H.reshape(H.shape[0] // 8, 8 * emb_k)
    U, V = _make_sc_gather(batch, emb_k)(
        u_idx >> 3, u_idx & 7, v_idx >> 3, v_idx & 7, wp, hp
    )
    a1 = _block_diag8(lin1_w[:, :emb_k].T)
    a2 = _block_diag8(lin1_w[:, emb_k:].T)
    b = jnp.tile(lin1_b, 8).reshape(1, 8 * emb_k)
    w2 = _block_diag8(lin2_w.T)
    out = _make_mlp(batch // 8, 8 * emb_k, 512)(U, V, a1, a2, b, w2)
    return out.reshape(batch, 1)


# speedup vs baseline: 4.8802x; 4.8326x over previous
import functools

import jax
import jax.numpy as jnp
from jax import lax
from jax.experimental import pallas as pl
from jax.experimental.pallas import tpu as pltpu
from jax.experimental.pallas import tpu_sc as plsc

NC = 2
NS = 16
NW = NC * NS
GRP = 16


def _sc_gather_body(bpw, uhi_hbm, ulo_hbm, vhi_hbm, vlo_hbm, wt_hbm, ht_hbm,
                    u_out, v_out, uhi_v, ulo_v, vhi_v, vlo_v, slots_v, outt_v,
                    sem):
    wid = lax.axis_index("s") * NC + lax.axis_index("c")
    base = wid * bpw
    pltpu.sync_copy(uhi_hbm.at[pl.ds(base, bpw)], uhi_v)
    pltpu.sync_copy(ulo_hbm.at[pl.ds(base, bpw)], ulo_v)
    pltpu.sync_copy(vhi_hbm.at[pl.ds(base, bpw)], vhi_v)
    pltpu.sync_copy(vlo_hbm.at[pl.ds(base, bpw)], vlo_v)

    j_iota = lax.iota(jnp.int32, GRP)

    def do_table(hi_v, lo_v, tab_hbm, out_hbm):
        def sg_body(sg, _):
            for g in range(8):
                off = sg * 128 + g * GRP
                c_vec = hi_v[pl.ds(off, GRP)]
                l_vec = lo_v[pl.ds(off, GRP)]
                copies = []
                for s in range(GRP):
                    cb = pl.multiple_of(c_vec[s] * 128, 128)
                    copies.append(pltpu.async_copy(
                        tab_hbm.at[:, pl.ds(cb, 128)], slots_v.at[s], sem))
                for cp in copies:
                    cp.wait()
                for k in range(16):
                    vals = plsc.load_gather(
                        slots_v, [j_iota, jnp.full((GRP,), k, jnp.int32),
                                  l_vec])
                    outt_v[k, pl.ds(g * GRP, GRP)] = vals
            colbase = pl.multiple_of(base + sg * 128, 128)
            pltpu.sync_copy(outt_v, out_hbm.at[:, pl.ds(colbase, 128)])
            return ()

        lax.fori_loop(0, bpw // 128, sg_body, ())

    do_table(uhi_v, ulo_v, wt_hbm, u_out)
    do_table(vhi_v, vlo_v, ht_hbm, v_out)


@functools.lru_cache(maxsize=None)
def _make_sc_gather(batch, emb_k):
    bpw = batch // NW
    mesh = plsc.VectorSubcoreMesh(core_axis_name="c", subcore_axis_name="s")
    return pl.kernel(
        functools.partial(_sc_gather_body, bpw),
        mesh=mesh,
        out_type=[
            jax.ShapeDtypeStruct((emb_k, batch), jnp.float32),
            jax.ShapeDtypeStruct((emb_k, batch), jnp.float32),
        ],
        scratch_types=[
            pltpu.VMEM((bpw,), jnp.int32),
            pltpu.VMEM((bpw,), jnp.int32),
            pltpu.VMEM((bpw,), jnp.int32),
            pltpu.VMEM((bpw,), jnp.int32),
            pltpu.VMEM((GRP, emb_k, 128), jnp.float32),
            pltpu.VMEM((emb_k, 128), jnp.float32),
            pltpu.SemaphoreType.DMA,
        ],
        compiler_params=pltpu.CompilerParams(needs_layout_passes=False),
    )


def _mlp_body(u_ref, v_ref, a1_ref, a2_ref, b_ref, w2_ref, o_ref):
    h = jnp.dot(a1_ref[...], u_ref[...], preferred_element_type=jnp.float32)
    h = h + jnp.dot(a2_ref[...], v_ref[...], preferred_element_type=jnp.float32)
    h = jnp.maximum(h + b_ref[...], 0.0)
    o_ref[...] = jnp.dot(w2_ref[...], h, preferred_element_type=jnp.float32)


@functools.lru_cache(maxsize=None)
def _make_mlp(batch, emb_k, blk):
    grid = (batch // blk,)
    return pl.pallas_call(
        _mlp_body,
        grid=grid,
        in_specs=[
            pl.BlockSpec((emb_k, blk), lambda i: (0, i)),
            pl.BlockSpec((emb_k, blk), lambda i: (0, i)),
            pl.BlockSpec((emb_k, emb_k), lambda i: (0, 0)),
            pl.BlockSpec((emb_k, emb_k), lambda i: (0, 0)),
            pl.BlockSpec((emb_k, 1), lambda i: (0, 0)),
            pl.BlockSpec((1, emb_k), lambda i: (0, 0)),
        ],
        out_specs=pl.BlockSpec((1, blk), lambda i: (0, i)),
        out_shape=jax.ShapeDtypeStruct((1, batch), jnp.float32),
    )


def kernel(x, W, H, lin1_w, lin1_b, lin2_w):
    batch = x.shape[0]
    emb_k = W.shape[1]
    u_idx = x[:, 0].astype(jnp.int32)
    v_idx = x[:, 1].astype(jnp.int32)
    U, V = _make_sc_gather(batch, emb_k)(
        u_idx >> 7, u_idx & 127, v_idx >> 7, v_idx & 127, W.T, H.T
    )
    a1 = lin1_w[:, :emb_k]
    a2 = lin1_w[:, emb_k:]
    b = lin1_b.reshape(emb_k, 1)
    out = _make_mlp(batch, emb_k, 2048)(U, V, a1, a2, b, lin2_w)
    return out.reshape(batch, 1)


# speedup vs baseline: 5.5632x; 1.1399x over previous
import functools

import jax
import jax.numpy as jnp
from jax import lax
from jax.experimental import pallas as pl
from jax.experimental.pallas import tpu as pltpu
from jax.experimental.pallas import tpu_sc as plsc

NC = 2
NS = 16
NW = NC * NS
GRP = 16


def _sc_gather_body(bpw, uhi_hbm, ulo_hbm, vhi_hbm, vlo_hbm, wt_hbm, ht_hbm,
                    u_out, v_out, uhi_v, ulo_v, vhi_v, vlo_v, slots_v, outt_v,
                    sem):
    wid = lax.axis_index("s") * NC + lax.axis_index("c")
    base = wid * bpw
    pltpu.sync_copy(uhi_hbm.at[pl.ds(base, bpw)], uhi_v)
    pltpu.sync_copy(ulo_hbm.at[pl.ds(base, bpw)], ulo_v)
    pltpu.sync_copy(vhi_hbm.at[pl.ds(base, bpw)], vhi_v)
    pltpu.sync_copy(vlo_hbm.at[pl.ds(base, bpw)], vlo_v)

    j_iota = lax.iota(jnp.int32, GRP)

    def do_table(hi_v, lo_v, tab_hbm, out_hbm):
        bufs = (slots_v.at[0], slots_v.at[1])
        sems = (sem.at[0], sem.at[1])

        def fire(off, p):
            c_vec = hi_v[pl.ds(off, GRP)]
            for s in range(GRP):
                cb = pl.multiple_of(c_vec[s] * 128, 128)
                pltpu.async_copy(
                    tab_hbm.at[:, pl.ds(cb, 128)], bufs[p].at[s], sems[p])

        def drain_extract(g, off, p):
            l_vec = lo_v[pl.ds(off, GRP)]
            for s in range(GRP):
                pltpu.make_async_copy(
                    tab_hbm.at[:, pl.ds(0, 128)], bufs[p].at[s], sems[p]
                ).wait()
            for k in range(16):
                vals = plsc.load_gather(
                    bufs[p], [j_iota, jnp.full((GRP,), k, jnp.int32), l_vec])
                outt_v[k, pl.ds(g * GRP, GRP)] = vals

        def sg_body(sg, _):
            base_sg = sg * 128
            fire(base_sg, 0)
            for g in range(8):
                p = g & 1
                if g < 7:
                    fire(base_sg + (g + 1) * GRP, 1 - p)
                drain_extract(g, base_sg + g * GRP, p)
            colbase = pl.multiple_of(base + base_sg, 128)
            pltpu.sync_copy(outt_v, out_hbm.at[:, pl.ds(colbase, 128)])
            return ()

        lax.fori_loop(0, bpw // 128, sg_body, ())

    do_table(uhi_v, ulo_v, wt_hbm, u_out)
    do_table(vhi_v, vlo_v, ht_hbm, v_out)


@functools.lru_cache(maxsize=None)
def _make_sc_gather(batch, emb_k):
    bpw = batch // NW
    mesh = plsc.VectorSubcoreMesh(core_axis_name="c", subcore_axis_name="s")
    return pl.kernel(
        functools.partial(_sc_gather_body, bpw),
        mesh=mesh,
        out_type=[
            jax.ShapeDtypeStruct((emb_k, batch), jnp.float32),
            jax.ShapeDtypeStruct((emb_k, batch), jnp.float32),
        ],
        scratch_types=[
            pltpu.VMEM((bpw,), jnp.int32),
            pltpu.VMEM((bpw,), jnp.int32),
            pltpu.VMEM((bpw,), jnp.int32),
            pltpu.VMEM((bpw,), jnp.int32),
            pltpu.VMEM((2, GRP, emb_k, 128), jnp.float32),
            pltpu.VMEM((emb_k, 128), jnp.float32),
            pltpu.SemaphoreType.DMA((2,)),
        ],
        compiler_params=pltpu.CompilerParams(needs_layout_passes=False),
    )


def _mlp_body(u_ref, v_ref, a1_ref, a2_ref, b_ref, w2_ref, o_ref):
    h = jnp.dot(a1_ref[...], u_ref[...], preferred_element_type=jnp.float32)
    h = h + jnp.dot(a2_ref[...], v_ref[...], preferred_element_type=jnp.float32)
    h = jnp.maximum(h + b_ref[...], 0.0)
    o_ref[...] = jnp.dot(w2_ref[...], h, preferred_element_type=jnp.float32)


@functools.lru_cache(maxsize=None)
def _make_mlp(batch, emb_k, blk):
    grid = (batch // blk,)
    return pl.pallas_call(
        _mlp_body,
        grid=grid,
        in_specs=[
            pl.BlockSpec((emb_k, blk), lambda i: (0, i)),
            pl.BlockSpec((emb_k, blk), lambda i: (0, i)),
            pl.BlockSpec((emb_k, emb_k), lambda i: (0, 0)),
            pl.BlockSpec((emb_k, emb_k), lambda i: (0, 0)),
            pl.BlockSpec((emb_k, 1), lambda i: (0, 0)),
            pl.BlockSpec((1, emb_k), lambda i: (0, 0)),
        ],
        out_specs=pl.BlockSpec((1, blk), lambda i: (0, i)),
        out_shape=jax.ShapeDtypeStruct((1, batch), jnp.float32),
    )


def kernel(x, W, H, lin1_w, lin1_b, lin2_w):
    batch = x.shape[0]
    emb_k = W.shape[1]
    u_idx = x[:, 0].astype(jnp.int32)
    v_idx = x[:, 1].astype(jnp.int32)
    U, V = _make_sc_gather(batch, emb_k)(
        u_idx >> 7, u_idx & 127, v_idx >> 7, v_idx & 127, W.T, H.T
    )
    a1 = lin1_w[:, :emb_k]
    a2 = lin1_w[:, emb_k:]
    b = lin1_b.reshape(emb_k, 1)
    out = _make_mlp(batch, emb_k, 2048)(U, V, a1, a2, b, lin2_w)
    return out.reshape(batch, 1)


# speedup vs baseline: 5.7904x; 1.0408x over previous
import functools

import jax
import jax.numpy as jnp
from jax import lax
from jax.experimental import pallas as pl
from jax.experimental.pallas import tpu as pltpu
from jax.experimental.pallas import tpu_sc as plsc

NC = 2
NS = 16
NW = NC * NS
GRP = 16


def _sc_gather_body(bpw, xt_hbm, wt_hbm, ht_hbm, u_out, v_out,
                    uidx_v, vidx_v, slots_v, outt_v, sem, osem):
    wid = lax.axis_index("s") * NC + lax.axis_index("c")
    base = wid * bpw
    pltpu.sync_copy(xt_hbm.at[0, pl.ds(base, bpw)], uidx_v)
    pltpu.sync_copy(xt_hbm.at[1, pl.ds(base, bpw)], vidx_v)

    j_iota = lax.iota(jnp.int32, GRP)
    n_sg = bpw // 128

    def do_table(t, idx_v, tab_hbm, out_hbm):
        bufs = (slots_v.at[0], slots_v.at[1])
        sems = (sem.at[0], sem.at[1])

        def fire(off, p):
            c_vec = idx_v[pl.ds(off, GRP)] >> 7
            for s in range(GRP):
                cb = pl.multiple_of(c_vec[s] * 128, 128)
                pltpu.async_copy(
                    tab_hbm.at[:, pl.ds(cb, 128)], bufs[p].at[s], sems[p])

        def drain_extract(g, off, p, ot):
            l_vec = idx_v[pl.ds(off, GRP)] & 127
            for s in range(GRP):
                pltpu.make_async_copy(
                    tab_hbm.at[:, pl.ds(0, 128)], bufs[p].at[s], sems[p]
                ).wait()
            for k in range(16):
                vals = plsc.load_gather(
                    bufs[p], [j_iota, jnp.full((GRP,), k, jnp.int32), l_vec])
                ot[k, pl.ds(g * GRP, GRP)] = vals

        def sg_body(sg, _):
            base_sg = sg * 128
            op = sg & 1
            ot = outt_v.at[op]
            @pl.when(sg + t * n_sg >= 2)
            def _():
                pltpu.make_async_copy(
                    outt_v.at[op], out_hbm.at[:, pl.ds(0, 128)], osem
                ).wait()
            for g in range(8):
                p = g & 1
                nxt = base_sg + (g + 1) * GRP

                @pl.when(nxt < bpw)
                def _():
                    fire(nxt, 1 - p)

                drain_extract(g, base_sg + g * GRP, p, ot)
            colbase = pl.multiple_of(base + base_sg, 128)
            pltpu.async_copy(ot, out_hbm.at[:, pl.ds(colbase, 128)], osem)
            return ()

        fire(0, 0)
        lax.fori_loop(0, n_sg, sg_body, ())

    do_table(0, uidx_v, wt_hbm, u_out)
    do_table(1, vidx_v, ht_hbm, v_out)
    pltpu.make_async_copy(
        outt_v.at[0], v_out.at[:, pl.ds(0, 128)], osem).wait()
    pltpu.make_async_copy(
        outt_v.at[1], v_out.at[:, pl.ds(0, 128)], osem).wait()


@functools.lru_cache(maxsize=None)
def _make_sc_gather(batch, emb_k):
    bpw = batch // NW
    mesh = plsc.VectorSubcoreMesh(core_axis_name="c", subcore_axis_name="s")
    return pl.kernel(
        functools.partial(_sc_gather_body, bpw),
        mesh=mesh,
        out_type=[
            jax.ShapeDtypeStruct((emb_k, batch), jnp.float32),
            jax.ShapeDtypeStruct((emb_k, batch), jnp.float32),
        ],
        scratch_types=[
            pltpu.VMEM((bpw,), jnp.int32),
            pltpu.VMEM((bpw,), jnp.int32),
            pltpu.VMEM((2, GRP, emb_k, 128), jnp.float32),
            pltpu.VMEM((2, emb_k, 128), jnp.float32),
            pltpu.SemaphoreType.DMA((2,)),
            pltpu.SemaphoreType.DMA,
        ],
        compiler_params=pltpu.CompilerParams(needs_layout_passes=False),
    )


def _mlp_body(u_ref, v_ref, a1_ref, a2_ref, b_ref, w2_ref, o_ref):
    h = jnp.dot(a1_ref[...], u_ref[...], preferred_element_type=jnp.float32)
    h = h + jnp.dot(a2_ref[...], v_ref[...], preferred_element_type=jnp.float32)
    h = jnp.maximum(h + b_ref[...], 0.0)
    o_ref[...] = jnp.dot(w2_ref[...], h, preferred_element_type=jnp.float32)


@functools.lru_cache(maxsize=None)
def _make_mlp(batch, emb_k, blk):
    grid = (batch // blk,)
    return pl.pallas_call(
        _mlp_body,
        grid=grid,
        in_specs=[
            pl.BlockSpec((emb_k, blk), lambda i: (0, i)),
            pl.BlockSpec((emb_k, blk), lambda i: (0, i)),
            pl.BlockSpec((emb_k, emb_k), lambda i: (0, 0)),
            pl.BlockSpec((emb_k, emb_k), lambda i: (0, 0)),
            pl.BlockSpec((emb_k, 1), lambda i: (0, 0)),
            pl.BlockSpec((1, emb_k), lambda i: (0, 0)),
        ],
        out_specs=pl.BlockSpec((1, blk), lambda i: (0, i)),
        out_shape=jax.ShapeDtypeStruct((1, batch), jnp.float32),
    )


def kernel(x, W, H, lin1_w, lin1_b, lin2_w):
    batch = x.shape[0]
    emb_k = W.shape[1]
    U, V = _make_sc_gather(batch, emb_k)(x.T, W.T, H.T)
    a1 = lin1_w[:, :emb_k]
    a2 = lin1_w[:, emb_k:]
    b = lin1_b.reshape(emb_k, 1)
    out = _make_mlp(batch, emb_k, 2048)(U, V, a1, a2, b, lin2_w)
    return out.reshape(batch, 1)


# speedup vs baseline: 5.9686x; 1.0308x over previous
import functools

import jax
import jax.numpy as jnp
from jax import lax
from jax.experimental import pallas as pl
from jax.experimental.pallas import tpu as pltpu
from jax.experimental.pallas import tpu_sc as plsc

NC = 2
NS = 16
NW = NC * NS
GRP = 16


def _sc_gather_body(bpw, xt_hbm, wt_hbm, ht_hbm, u_out, v_out,
                    uidx_v, vidx_v, slots_v, outt_v, sem, osem):
    wid = lax.axis_index("s") * NC + lax.axis_index("c")
    base = wid * bpw
    pltpu.sync_copy(xt_hbm.at[0, pl.ds(base, bpw)], uidx_v)
    pltpu.sync_copy(xt_hbm.at[1, pl.ds(base, bpw)], vidx_v)

    j_iota = lax.iota(jnp.int32, GRP)
    n_sg = bpw // 128

    def do_table(t, idx_v, tab_hbm, out_hbm):
        bufs = (slots_v.at[0], slots_v.at[1])
        sems = (sem.at[0], sem.at[1])

        def fire(off, p):
            c_vec = idx_v[pl.ds(off, GRP)] >> 7
            for s in range(GRP):
                cb = pl.multiple_of(c_vec[s] * 128, 128)
                pltpu.async_copy(
                    tab_hbm.at[:, pl.ds(cb, 128)], bufs[p].at[s], sems[p])

        def drain_extract(g, off, p, ot):
            l_vec = idx_v[pl.ds(off, GRP)] & 127
            for s in range(GRP):
                pltpu.make_async_copy(
                    tab_hbm.at[:, pl.ds(0, 128)], bufs[p].at[s], sems[p]
                ).wait()
            gcol = pl.multiple_of(g * GRP, GRP)
            for k in range(16):
                vals = plsc.load_gather(
                    bufs[p], [j_iota, jnp.full((GRP,), k, jnp.int32), l_vec])
                ot[k, pl.ds(gcol, GRP)] = vals

        def sg_body(sg, _):
            base_sg = sg * 128
            op = sg & 1
            ot = outt_v.at[op]
            @pl.when(sg + t * n_sg >= 2)
            def _():
                pltpu.make_async_copy(
                    outt_v.at[op], out_hbm.at[:, pl.ds(0, 128)], osem
                ).wait()

            def grp_pair(i, _):
                for h in range(2):
                    g = 2 * i + h
                    off = base_sg + g * GRP
                    nxt = off + GRP

                    @pl.when(nxt < bpw)
                    def _():
                        fire(nxt, 1 - h)

                    drain_extract(g, off, h, ot)
                return ()

            lax.fori_loop(0, 4, grp_pair, ())
            colbase = pl.multiple_of(base + base_sg, 128)
            pltpu.async_copy(ot, out_hbm.at[:, pl.ds(colbase, 128)], osem)
            return ()

        fire(0, 0)
        lax.fori_loop(0, n_sg, sg_body, ())

    do_table(0, uidx_v, wt_hbm, u_out)
    do_table(1, vidx_v, ht_hbm, v_out)
    pltpu.make_async_copy(
        outt_v.at[0], v_out.at[:, pl.ds(0, 128)], osem).wait()
    pltpu.make_async_copy(
        outt_v.at[1], v_out.at[:, pl.ds(0, 128)], osem).wait()


@functools.lru_cache(maxsize=None)
def _make_sc_gather(batch, emb_k):
    bpw = batch // NW
    mesh = plsc.VectorSubcoreMesh(core_axis_name="c", subcore_axis_name="s")
    return pl.kernel(
        functools.partial(_sc_gather_body, bpw),
        mesh=mesh,
        out_type=[
            jax.ShapeDtypeStruct((emb_k, batch), jnp.float32),
            jax.ShapeDtypeStruct((emb_k, batch), jnp.float32),
        ],
        scratch_types=[
            pltpu.VMEM((bpw,), jnp.int32),
            pltpu.VMEM((bpw,), jnp.int32),
            pltpu.VMEM((2, GRP, emb_k, 128), jnp.float32),
            pltpu.VMEM((2, emb_k, 128), jnp.float32),
            pltpu.SemaphoreType.DMA((2,)),
            pltpu.SemaphoreType.DMA,
        ],
        compiler_params=pltpu.CompilerParams(needs_layout_passes=False),
    )


def _mlp_body(u_ref, v_ref, a1_ref, a2_ref, b_ref, w2_ref, o_ref):
    h = jnp.dot(a1_ref[...], u_ref[...], preferred_element_type=jnp.float32)
    h = h + jnp.dot(a2_ref[...], v_ref[...], preferred_element_type=jnp.float32)
    h = jnp.maximum(h + b_ref[...], 0.0)
    o_ref[...] = jnp.dot(w2_ref[...], h, preferred_element_type=jnp.float32)


@functools.lru_cache(maxsize=None)
def _make_mlp(batch, emb_k, blk):
    grid = (batch // blk,)
    return pl.pallas_call(
        _mlp_body,
        grid=grid,
        in_specs=[
            pl.BlockSpec((emb_k, blk), lambda i: (0, i)),
            pl.BlockSpec((emb_k, blk), lambda i: (0, i)),
            pl.BlockSpec((emb_k, emb_k), lambda i: (0, 0)),
            pl.BlockSpec((emb_k, emb_k), lambda i: (0, 0)),
            pl.BlockSpec((emb_k, 1), lambda i: (0, 0)),
            pl.BlockSpec((1, emb_k), lambda i: (0, 0)),
        ],
        out_specs=pl.BlockSpec((1, blk), lambda i: (0, i)),
        out_shape=jax.ShapeDtypeStruct((1, batch), jnp.float32),
    )


def kernel(x, W, H, lin1_w, lin1_b, lin2_w):
    batch = x.shape[0]
    emb_k = W.shape[1]
    U, V = _make_sc_gather(batch, emb_k)(x.T, W.T, H.T)
    a1 = lin1_w[:, :emb_k]
    a2 = lin1_w[:, emb_k:]
    b = lin1_b.reshape(emb_k, 1)
    out = _make_mlp(batch, emb_k, 2048)(U, V, a1, a2, b, lin2_w)
    return out.reshape(batch, 1)


# speedup vs baseline: 6.1052x; 1.0229x over previous
import functools

import jax
import jax.numpy as jnp
from jax import lax
from jax.experimental import pallas as pl
from jax.experimental.pallas import tpu as pltpu
from jax.experimental.pallas import tpu_sc as plsc

NC = 2
NS = 16
NW = NC * NS
GRP = 16


def _sc_gather_body(bpw, xt_hbm, wt_hbm, ht_hbm, u_out, v_out,
                    uidx_v, vidx_v, slots_v, outt_v, sem, osem):
    wid = lax.axis_index("s") * NC + lax.axis_index("c")
    base = wid * bpw
    pltpu.sync_copy(xt_hbm.at[0, pl.ds(base, bpw)], uidx_v)
    pltpu.sync_copy(xt_hbm.at[1, pl.ds(base, bpw)], vidx_v)

    j_iota = lax.iota(jnp.int32, GRP)
    n_sg = bpw // 128

    def do_table(t, idx_v, tab_hbm, out_hbm):
        bufs = (slots_v.at[0], slots_v.at[1])
        sems = (sem.at[0], sem.at[1])

        def fire(off, p):
            c_vec = idx_v[pl.ds(off, GRP)] >> 7
            for s in range(GRP):
                cb = pl.multiple_of(c_vec[s] * 128, 128)
                pltpu.async_copy(
                    tab_hbm.at[:, pl.ds(cb, 128)], bufs[p].at[s], sems[p])

        def drain_extract(g, off, p, ot):
            l_vec = idx_v[pl.ds(off, GRP)] & 127
            for s in range(GRP):
                pltpu.make_async_copy(
                    tab_hbm.at[:, pl.ds(0, 128)], bufs[p].at[s], sems[p]
                ).wait()
            gcol = pl.multiple_of(g * GRP, GRP)
            for k in range(16):
                vals = plsc.load_gather(
                    bufs[p], [j_iota, jnp.full((GRP,), k, jnp.int32), l_vec])
                ot[k, pl.ds(gcol, GRP)] = vals

        def sg_body(sg, _):
            base_sg = sg * 128
            op = sg & 1
            ot = outt_v.at[op]
            @pl.when(sg + t * n_sg >= 2)
            def _():
                pltpu.make_async_copy(
                    outt_v.at[op], out_hbm.at[:, pl.ds(0, 128)], osem
                ).wait()

            def grp_pair(i, _):
                for h in range(2):
                    g = 2 * i + h
                    off = base_sg + g * GRP
                    nxt = off + GRP

                    @pl.when(nxt < bpw)
                    def _():
                        fire(nxt, 1 - h)

                    drain_extract(g, off, h, ot)
                return ()

            lax.fori_loop(0, 4, grp_pair, ())
            colbase = pl.multiple_of(base + base_sg, 128)
            pltpu.async_copy(ot, out_hbm.at[:, pl.ds(colbase, 128)], osem)
            return ()

        fire(0, 0)
        lax.fori_loop(0, n_sg, sg_body, ())

    do_table(0, uidx_v, wt_hbm, u_out)
    do_table(1, vidx_v, ht_hbm, v_out)
    pltpu.make_async_copy(
        outt_v.at[0], v_out.at[:, pl.ds(0, 128)], osem).wait()
    pltpu.make_async_copy(
        outt_v.at[1], v_out.at[:, pl.ds(0, 128)], osem).wait()


@functools.lru_cache(maxsize=None)
def _make_sc_gather(batch, emb_k):
    bpw = batch // NW
    mesh = plsc.VectorSubcoreMesh(core_axis_name="c", subcore_axis_name="s")
    return pl.kernel(
        functools.partial(_sc_gather_body, bpw),
        mesh=mesh,
        out_type=[
            jax.ShapeDtypeStruct((emb_k, batch), jnp.float32),
            jax.ShapeDtypeStruct((emb_k, batch), jnp.float32),
        ],
        scratch_types=[
            pltpu.VMEM((bpw,), jnp.int32),
            pltpu.VMEM((bpw,), jnp.int32),
            pltpu.VMEM((2, GRP, emb_k, 128), jnp.float32),
            pltpu.VMEM((2, emb_k, 128), jnp.float32),
            pltpu.SemaphoreType.DMA((2,)),
            pltpu.SemaphoreType.DMA,
        ],
        compiler_params=pltpu.CompilerParams(needs_layout_passes=False),
    )


def _mlp_body(u_ref, v_ref, a1_ref, a2_ref, b_ref, w2_ref, o_ref):
    h = jnp.dot(a1_ref[...], u_ref[...], preferred_element_type=jnp.float32)
    h = h + jnp.dot(a2_ref[...], v_ref[...], preferred_element_type=jnp.float32)
    h = jnp.maximum(h + b_ref[...], 0.0)
    o_ref[...] = jnp.dot(w2_ref[...], h, preferred_element_type=jnp.float32)


@functools.lru_cache(maxsize=None)
def _make_mlp(batch, emb_k, blk):
    grid = (batch // blk,)
    return pl.pallas_call(
        _mlp_body,
        grid=grid,
        in_specs=[
            pl.BlockSpec((emb_k, blk), lambda i: (0, i)),
            pl.BlockSpec((emb_k, blk), lambda i: (0, i)),
            pl.BlockSpec((emb_k, emb_k), lambda i: (0, 0)),
            pl.BlockSpec((emb_k, emb_k), lambda i: (0, 0)),
            pl.BlockSpec((emb_k, 1), lambda i: (0, 0)),
            pl.BlockSpec((1, emb_k), lambda i: (0, 0)),
        ],
        out_specs=pl.BlockSpec((1, blk), lambda i: (0, i)),
        out_shape=jax.ShapeDtypeStruct((1, batch), jnp.float32),
    )


def kernel(x, W, H, lin1_w, lin1_b, lin2_w):
    batch = x.shape[0]
    emb_k = W.shape[1]
    U, V = _make_sc_gather(batch, emb_k)(x.T, W.T, H.T)
    a1 = lin1_w[:, :emb_k]
    a2 = lin1_w[:, emb_k:]
    b = lin1_b.reshape(emb_k, 1)
    out = _make_mlp(batch, emb_k, 4096)(U, V, a1, a2, b, lin2_w)
    return out.reshape(batch, 1)


# speedup vs baseline: 6.1469x; 1.0068x over previous
import functools

import jax
import jax.numpy as jnp
from jax import lax
from jax.experimental import pallas as pl
from jax.experimental.pallas import tpu as pltpu
from jax.experimental.pallas import tpu_sc as plsc

NC = 2
NS = 16
NW = NC * NS
GRP = 16


def _sc_gather_body(bpw, xt_hbm, wt_hbm, ht_hbm, u_out, v_out,
                    uidx_v, vidx_v, slots_v, outt_v, sem, osem):
    wid = lax.axis_index("s") * NC + lax.axis_index("c")
    base = wid * bpw
    pltpu.sync_copy(xt_hbm.at[0, pl.ds(base, bpw)], uidx_v)
    pltpu.sync_copy(xt_hbm.at[1, pl.ds(base, bpw)], vidx_v)

    j_iota = lax.iota(jnp.int32, GRP)
    n_sg = bpw // 128

    def do_table(t, idx_v, tab_hbm, out_hbm):
        bufs = (slots_v.at[0], slots_v.at[1])
        sems = (sem.at[0], sem.at[1])

        def fire(off, p):
            c_vec = idx_v[pl.ds(off, GRP)] >> 7
            for s in range(GRP):
                cb = pl.multiple_of(c_vec[s] * 128, 128)
                pltpu.async_copy(
                    tab_hbm.at[:, pl.ds(cb, 128)], bufs[p].at[s], sems[p])

        def drain_extract(g, off, p, ot):
            l_vec = idx_v[pl.ds(off, GRP)] & 127
            for s in range(GRP):
                pltpu.make_async_copy(
                    tab_hbm.at[:, pl.ds(0, 128)], bufs[p].at[s], sems[p]
                ).wait()
            gcol = pl.multiple_of(g * GRP, GRP)
            for k in range(16):
                vals = plsc.load_gather(
                    bufs[p], [j_iota, jnp.full((GRP,), k, jnp.int32), l_vec])
                ot[k, pl.ds(gcol, GRP)] = vals

        def sg_body(sg, _):
            base_sg = sg * 128
            op = sg & 1
            ot = outt_v.at[op]
            @pl.when(sg + t * n_sg >= 2)
            def _():
                pltpu.make_async_copy(
                    outt_v.at[op], out_hbm.at[:, pl.ds(0, 128)], osem
                ).wait()

            def grp_pair(i, _):
                for h in range(2):
                    g = 2 * i + h
                    off = base_sg + g * GRP
                    nxt = off + GRP

                    @pl.when(nxt < bpw)
                    def _():
                        fire(nxt, 1 - h)

                    drain_extract(g, off, h, ot)
                return ()

            lax.fori_loop(0, 4, grp_pair, ())
            colbase = pl.multiple_of(base + base_sg, 128)
            pltpu.async_copy(ot, out_hbm.at[:, pl.ds(colbase, 128)], osem)
            return ()

        fire(0, 0)
        lax.fori_loop(0, n_sg, sg_body, ())

    do_table(0, uidx_v, wt_hbm, u_out)
    do_table(1, vidx_v, ht_hbm, v_out)
    pltpu.make_async_copy(
        outt_v.at[0], v_out.at[:, pl.ds(0, 128)], osem).wait()
    pltpu.make_async_copy(
        outt_v.at[1], v_out.at[:, pl.ds(0, 128)], osem).wait()


@functools.lru_cache(maxsize=None)
def _make_sc_gather(batch, emb_k):
    bpw = batch // NW
    mesh = plsc.VectorSubcoreMesh(core_axis_name="c", subcore_axis_name="s")
    return pl.kernel(
        functools.partial(_sc_gather_body, bpw),
        mesh=mesh,
        out_type=[
            jax.ShapeDtypeStruct((emb_k, batch), jnp.float32),
            jax.ShapeDtypeStruct((emb_k, batch), jnp.float32),
        ],
        scratch_types=[
            pltpu.VMEM((bpw,), jnp.int32),
            pltpu.VMEM((bpw,), jnp.int32),
            pltpu.VMEM((2, GRP, emb_k, 128), jnp.float32),
            pltpu.VMEM((2, emb_k, 128), jnp.float32),
            pltpu.SemaphoreType.DMA((2,)),
            pltpu.SemaphoreType.DMA,
        ],
        compiler_params=pltpu.CompilerParams(needs_layout_passes=False),
    )


def _mlp_body(u_ref, v_ref, a1_ref, a2_ref, b_ref, w2_ref, o_ref):
    h = jnp.dot(a1_ref[...], u_ref[...], preferred_element_type=jnp.float32)
    h = h + jnp.dot(a2_ref[...], v_ref[...], preferred_element_type=jnp.float32)
    h = jnp.maximum(h + b_ref[...], 0.0)
    o_ref[...] = jnp.dot(w2_ref[...], h, preferred_element_type=jnp.float32)


@functools.lru_cache(maxsize=None)
def _make_mlp(batch, emb_k, blk):
    grid = (batch // blk,)
    return pl.pallas_call(
        _mlp_body,
        grid=grid,
        in_specs=[
            pl.BlockSpec((emb_k, blk), lambda i: (0, i)),
            pl.BlockSpec((emb_k, blk), lambda i: (0, i)),
            pl.BlockSpec((emb_k, emb_k), lambda i: (0, 0)),
            pl.BlockSpec((emb_k, emb_k), lambda i: (0, 0)),
            pl.BlockSpec((emb_k, 1), lambda i: (0, 0)),
            pl.BlockSpec((1, emb_k), lambda i: (0, 0)),
        ],
        out_specs=pl.BlockSpec((1, blk), lambda i: (0, i)),
        out_shape=jax.ShapeDtypeStruct((1, batch), jnp.float32),
    )


def kernel(x, W, H, lin1_w, lin1_b, lin2_w):
    batch = x.shape[0]
    emb_k = W.shape[1]
    U, V = _make_sc_gather(batch, emb_k)(x.T, W.T, H.T)
    a1 = lin1_w[:, :emb_k]
    a2 = lin1_w[:, emb_k:]
    b = lin1_b.reshape(emb_k, 1)
    out = _make_mlp(batch, emb_k, 8192)(U, V, a1, a2, b, lin2_w)
    return out.reshape(batch, 1)
